# Initial kernel scaffold; baseline (speedup 1.0000x reference)
import jax
import jax.numpy as jnp
from jax.experimental import pallas as pl


def kernel(xyz1, xyz2, features1, features2, W1, b1, W2, b2):
    raise NotImplementedError("write your pallas kernel here")


# speedup vs baseline: 8.1058x; 8.1058x over previous
import functools

import jax
import jax.numpy as jnp
from jax import lax
from jax.experimental import pallas as pl
from jax.experimental.pallas import tpu as pltpu
from jax.experimental.pallas import tpu_sc as plsc

B, N1, N2, C1, C2 = 8, 4096, 1024, 128, 256
BN = B * N1
TILE = 512
TILE3 = 1024
NW = 32
PW = BN // NW
CH = 64


def _topk_body(x1_ref, x2t_ref, idx_ref, w_ref):
    b = pl.program_id(0)
    x1 = x1_ref[0]
    x2t = x2t_ref[0]
    d = None
    for c in range(3):
        term = x1[:, c:c + 1] - x2t[c:c + 1, :]
        sq = term * term
        d = sq if d is None else d + sq
    iota = lax.broadcasted_iota(jnp.int32, (TILE, N2), 1)
    cur = d
    ims, ms = [], []
    for k in range(3):
        m = jnp.min(cur, axis=1, keepdims=True)
        im = jnp.min(jnp.where(cur == m, iota, N2), axis=1, keepdims=True)
        ms.append(m)
        ims.append(im)
        if k < 2:
            cur = jnp.where(iota == im, jnp.float32(jnp.inf), cur)
    dist3 = jnp.maximum(jnp.concatenate(ms, axis=1), 1e-10)
    inv = 1.0 / dist3
    w3 = inv / jnp.sum(inv, axis=1, keepdims=True)
    idx3 = jnp.concatenate(ims, axis=1) + b * N2
    idx_ref[0] = idx3
    w_ref[0] = w3


def _topk_stage(xyz1, xyz2t):
    return pl.pallas_call(
        _topk_body,
        grid=(B, N1 // TILE),
        in_specs=[
            pl.BlockSpec((1, TILE, 3), lambda b, t: (b, t, 0)),
            pl.BlockSpec((1, 3, N2), lambda b, t: (b, 0, 0)),
        ],
        out_specs=[
            pl.BlockSpec((1, TILE, 3), lambda b, t: (b, t, 0)),
            pl.BlockSpec((1, TILE, 3), lambda b, t: (b, t, 0)),
        ],
        out_shape=[
            jax.ShapeDtypeStruct((B, N1, 3), jnp.int32),
            jax.ShapeDtypeStruct((B, N1, 3), jnp.float32),
        ],
    )(xyz1, xyz2t)


def _sc_interp(idx_pl, w_pl, table):
    mesh = plsc.VectorSubcoreMesh(core_axis_name="c", subcore_axis_name="s")

    @functools.partial(
        pl.kernel,
        mesh=mesh,
        compiler_params=pltpu.CompilerParams(use_tc_tiling_on_sc=False,
                                             needs_layout_passes=False),
        out_type=jax.ShapeDtypeStruct((BN, C2), jnp.float32),
        scratch_types=[
            pltpu.VMEM((CH,), jnp.int32),
            pltpu.VMEM((CH,), jnp.int32),
            pltpu.VMEM((CH,), jnp.int32),
            pltpu.VMEM((CH,), jnp.float32),
            pltpu.VMEM((CH,), jnp.float32),
            pltpu.VMEM((CH,), jnp.float32),
            pltpu.VMEM((CH, C2), jnp.float32),
            pltpu.VMEM((CH, C2), jnp.float32),
            pltpu.VMEM((CH, C2), jnp.float32),
            pltpu.VMEM((CH, C2), jnp.float32),
            pltpu.SemaphoreType.DMA,
        ],
    )
    def k(idx_hbm, w_hbm, table_hbm, out_hbm,
          i0, i1, i2, w0, w1, w2, r0, r1, r2, ov, sem):
        wid = lax.axis_index("s") * 2 + lax.axis_index("c")
        base_w = wid * PW
        lane = lax.iota(jnp.int32, 16)

        def chunk(ci, _):
            base = base_w + ci * CH
            pltpu.sync_copy(idx_hbm.at[0, pl.ds(base, CH)], i0)
            pltpu.sync_copy(idx_hbm.at[1, pl.ds(base, CH)], i1)
            pltpu.sync_copy(idx_hbm.at[2, pl.ds(base, CH)], i2)
            pltpu.sync_copy(w_hbm.at[0, pl.ds(base, CH)], w0)
            pltpu.sync_copy(w_hbm.at[1, pl.ds(base, CH)], w1)
            pltpu.sync_copy(w_hbm.at[2, pl.ds(base, CH)], w2)
            pltpu.async_copy(table_hbm.at[i0], r0, sem).wait()
            pltpu.async_copy(table_hbm.at[i1], r1, sem).wait()
            pltpu.async_copy(table_hbm.at[i2], r2, sem).wait()

            def group(g, _):
                prow = g * 16 + lane
                a0 = w0[pl.ds(g * 16, 16)]
                a1 = w1[pl.ds(g * 16, 16)]
                a2 = w2[pl.ds(g * 16, 16)]
                for c in range(C2):
                    cvec = jnp.full((16,), c, jnp.int32)
                    acc = plsc.load_gather(r0, [prow, cvec]) * a0
                    acc = acc + plsc.load_gather(r1, [prow, cvec]) * a1
                    acc = acc + plsc.load_gather(r2, [prow, cvec]) * a2
                    plsc.store_scatter(ov, [prow, cvec], acc)
                return 0

            lax.fori_loop(0, CH // 16, group, 0)
            pltpu.sync_copy(ov, out_hbm.at[pl.ds(base, CH)])
            return 0

        lax.fori_loop(0, PW // CH, chunk, 0)

    return k(idx_pl, w_pl, table)


def _mlp_body(interp_ref, f1_ref, w1a_ref, w1b_ref, b1_ref, w2_ref, b2_ref, out_ref):
    h = jnp.dot(interp_ref[...], w1a_ref[...], preferred_element_type=jnp.float32)
    h = h + jnp.dot(f1_ref[...], w1b_ref[...], preferred_element_type=jnp.float32)
    h = jnp.maximum(h + b1_ref[...], 0.0)
    o = jnp.dot(h, w2_ref[...], preferred_element_type=jnp.float32)
    out_ref[...] = jnp.maximum(o + b2_ref[...], 0.0)


def _mlp_stage(interp, f1, w1a, w1b, b1, w2, b2):
    return pl.pallas_call(
        _mlp_body,
        grid=(BN // TILE3,),
        in_specs=[
            pl.BlockSpec((TILE3, C2), lambda t: (t, 0)),
            pl.BlockSpec((TILE3, C1), lambda t: (t, 0)),
            pl.BlockSpec((C2, 256), lambda t: (0, 0)),
            pl.BlockSpec((C1, 256), lambda t: (0, 0)),
            pl.BlockSpec((1, 256), lambda t: (0, 0)),
            pl.BlockSpec((256, 256), lambda t: (0, 0)),
            pl.BlockSpec((1, 256), lambda t: (0, 0)),
        ],
        out_specs=pl.BlockSpec((TILE3, 256), lambda t: (t, 0)),
        out_shape=jax.ShapeDtypeStruct((BN, 256), jnp.float32),
    )(interp, f1, w1a, w1b, b1, w2, b2)


def kernel(xyz1, xyz2, features1, features2, W1, b1, W2, b2):
    xyz2t = jnp.transpose(xyz2, (0, 2, 1))
    idx3, w3 = _topk_stage(xyz1, xyz2t)
    idx_pl = jnp.transpose(idx3.reshape(BN, 3), (1, 0))
    w_pl = jnp.transpose(w3.reshape(BN, 3), (1, 0))
    table = features2.reshape(B * N2, C2)
    interp = _sc_interp(idx_pl, w_pl, table)
    out = _mlp_stage(interp, features1.reshape(BN, C1),
                     W1[:C2], W1[C2:], b1.reshape(1, 256),
                     W2, b2.reshape(1, 256))
    return out.reshape(B, N1, 256)


# speedup vs baseline: 20.9001x; 2.5784x over previous
import functools

import jax
import jax.numpy as jnp
from jax import lax
from jax.experimental import pallas as pl
from jax.experimental.pallas import tpu as pltpu
from jax.experimental.pallas import tpu_sc as plsc

B, N1, N2, C1, C2 = 8, 4096, 1024, 128, 256
BN = B * N1
TILE = 512
TILE3 = 1024
NW = 32
PW = BN // NW
CH = 64


def _topk_body(x1_ref, x2t_ref, idx_ref, w_ref):
    b = pl.program_id(0)
    x1 = x1_ref[0]
    x2t = x2t_ref[0]
    d = None
    for c in range(3):
        term = x1[:, c:c + 1] - x2t[c:c + 1, :]
        sq = term * term
        d = sq if d is None else d + sq
    iota = lax.broadcasted_iota(jnp.int32, (TILE, N2), 1)
    cur = d
    ims, ms = [], []
    for k in range(3):
        m = jnp.min(cur, axis=1, keepdims=True)
        im = jnp.min(jnp.where(cur == m, iota, N2), axis=1, keepdims=True)
        ms.append(m)
        ims.append(im)
        if k < 2:
            cur = jnp.where(iota == im, jnp.float32(jnp.inf), cur)
    dist3 = jnp.maximum(jnp.concatenate(ms, axis=1), 1e-10)
    inv = 1.0 / dist3
    w3 = inv / jnp.sum(inv, axis=1, keepdims=True)
    idx3 = jnp.concatenate(ims, axis=1) + b * N2
    idx_ref[0] = idx3
    w_ref[0] = w3


def _topk_stage(xyz1, xyz2t):
    return pl.pallas_call(
        _topk_body,
        grid=(B, N1 // TILE),
        in_specs=[
            pl.BlockSpec((1, TILE, 3), lambda b, t: (b, t, 0)),
            pl.BlockSpec((1, 3, N2), lambda b, t: (b, 0, 0)),
        ],
        out_specs=[
            pl.BlockSpec((1, TILE, 3), lambda b, t: (b, t, 0)),
            pl.BlockSpec((1, TILE, 3), lambda b, t: (b, t, 0)),
        ],
        out_shape=[
            jax.ShapeDtypeStruct((B, N1, 3), jnp.int32),
            jax.ShapeDtypeStruct((B, N1, 3), jnp.float32),
        ],
    )(xyz1, xyz2t)


def _sc_interp(idx_pl, w_pl, table):
    mesh = plsc.VectorSubcoreMesh(core_axis_name="c", subcore_axis_name="s")

    @functools.partial(
        pl.kernel,
        mesh=mesh,
        compiler_params=pltpu.CompilerParams(use_tc_tiling_on_sc=False,
                                             needs_layout_passes=False),
        out_type=jax.ShapeDtypeStruct((BN, C2), jnp.float32),
        scratch_types=[
            pltpu.VMEM((3, PW), jnp.int32),
            pltpu.VMEM((3, PW + 16), jnp.float32),
            pltpu.VMEM((CH, C2), jnp.float32),
            pltpu.VMEM((CH, C2), jnp.float32),
            pltpu.VMEM((CH, C2), jnp.float32),
            pltpu.VMEM((CH, C2), jnp.float32),
            pltpu.SemaphoreType.DMA,
        ],
    )
    def k(idx_hbm, w_hbm, table_hbm, out_hbm,
          iv, wv, r0, r1, r2, ov, sem):
        wid = lax.axis_index("s") * 2 + lax.axis_index("c")
        base_w = wid * PW
        zero16 = jnp.zeros((16, 1), jnp.int32)
        dnums = lax.GatherDimensionNumbers(
            offset_dims=(), collapsed_slice_dims=(0,), start_index_map=(0,))

        def bcast0(vec):
            return lax.gather(vec, zero16, dnums, slice_sizes=(1,),
                              mode=lax.GatherScatterMode.PROMISE_IN_BOUNDS)
        pltpu.sync_copy(idx_hbm.at[:, pl.ds(base_w, PW)], iv)
        pltpu.sync_copy(w_hbm.at[:, pl.ds(base_w, PW)],
                        wv.at[:, pl.ds(0, PW)])

        def chunk(ci, _):
            off = ci * CH
            cp0 = pltpu.async_copy(table_hbm.at[iv.at[0, pl.ds(off, CH)]], r0, sem)
            cp1 = pltpu.async_copy(table_hbm.at[iv.at[1, pl.ds(off, CH)]], r1, sem)
            cp2 = pltpu.async_copy(table_hbm.at[iv.at[2, pl.ds(off, CH)]], r2, sem)
            cp0.wait()
            cp1.wait()
            cp2.wait()

            @plsc.parallel_loop(0, CH, unroll=2)
            def point(p):
                b0 = bcast0(wv[0, pl.ds(off + p, 16)])
                b1 = bcast0(wv[1, pl.ds(off + p, 16)])
                b2 = bcast0(wv[2, pl.ds(off + p, 16)])
                for c in range(C2 // 16):
                    s = pl.ds(c * 16, 16)
                    acc = r0[p, s] * b0 + r1[p, s] * b1 + r2[p, s] * b2
                    ov[p, s] = acc

            pltpu.sync_copy(ov, out_hbm.at[pl.ds(base_w + off, CH)])
            return 0

        lax.fori_loop(0, PW // CH, chunk, 0)

    return k(idx_pl, w_pl, table)


def _mlp_body(interp_ref, f1_ref, w1a_ref, w1b_ref, b1_ref, w2_ref, b2_ref, out_ref):
    h = jnp.dot(interp_ref[...], w1a_ref[...], preferred_element_type=jnp.float32)
    h = h + jnp.dot(f1_ref[...], w1b_ref[...], preferred_element_type=jnp.float32)
    h = jnp.maximum(h + b1_ref[...], 0.0)
    o = jnp.dot(h, w2_ref[...], preferred_element_type=jnp.float32)
    out_ref[...] = jnp.maximum(o + b2_ref[...], 0.0)


def _mlp_stage(interp, f1, w1a, w1b, b1, w2, b2):
    return pl.pallas_call(
        _mlp_body,
        grid=(BN // TILE3,),
        in_specs=[
            pl.BlockSpec((TILE3, C2), lambda t: (t, 0)),
            pl.BlockSpec((TILE3, C1), lambda t: (t, 0)),
            pl.BlockSpec((C2, 256), lambda t: (0, 0)),
            pl.BlockSpec((C1, 256), lambda t: (0, 0)),
            pl.BlockSpec((1, 256), lambda t: (0, 0)),
            pl.BlockSpec((256, 256), lambda t: (0, 0)),
            pl.BlockSpec((1, 256), lambda t: (0, 0)),
        ],
        out_specs=pl.BlockSpec((TILE3, 256), lambda t: (t, 0)),
        out_shape=jax.ShapeDtypeStruct((BN, 256), jnp.float32),
    )(interp, f1, w1a, w1b, b1, w2, b2)


def kernel(xyz1, xyz2, features1, features2, W1, b1, W2, b2):
    xyz2t = jnp.transpose(xyz2, (0, 2, 1))
    idx3, w3 = _topk_stage(xyz1, xyz2t)
    idx_pl = jnp.transpose(idx3.reshape(BN, 3), (1, 0))
    w_pl = jnp.transpose(w3.reshape(BN, 3), (1, 0))
    table = features2.reshape(B * N2, C2)
    interp = _sc_interp(idx_pl, w_pl, table)
    out = _mlp_stage(interp, features1.reshape(BN, C1),
                     W1[:C2], W1[C2:], b1.reshape(1, 256),
                     W2, b2.reshape(1, 256))
    return out.reshape(B, N1, 256)
